# baseline (speedup 1.0000x reference)
import numpy as np

import jax
import jax.numpy as jnp
from jax.experimental import pallas as pl
from jax.experimental.pallas import tpu as pltpu

_EPS = 1e-5
_BT = 1024
_NP = 128
_NC = 16

_R_T1, _R_U2, _R_FW1, _R_FW2, _R_FW3, _R_FB3 = 0, 384, 704, 832, 960, 976
_WROWS, _WCOLS = 984, 640
_R_S1, _R_S2, _R_FB1, _R_FB2 = 0, 96, 176, 304
_BROWS = 432
_W_FC1, _W_FC2, _W_FC3, _W_C2, _W_C1 = 0, 120, 204, 214, 230


def _patterns():
    G1 = np.zeros((25, 4 * 192), np.float32)
    G2 = np.zeros((152, 4 * 576), np.float32)
    for kh in range(5):
        for kw in range(5):
            for dr in range(2):
                for dc in range(2):
                    d = dr * 2 + dc
                    G1[kh * 5 + kw, d * 192 + (dr + kh) * 32 + dc + kw] = 1.0
                    for ci in range(6):
                        G2[ci * 25 + kh * 5 + kw,
                           d * 576 + (dr + kh) * 96 + (dc + kw) * 6 + ci] = 1.0

    S1 = np.zeros((400, 400), np.float32)
    for ii in range(5):
        for j2 in range(5):
            for c2 in range(16):
                S1[c2 * 25 + ii * 5 + j2, ii * 80 + j2 * 16 + c2] = 1.0

    EP = np.zeros((96, 40), np.float32)
    for j in range(14):
        for c in range(6):
            EP[j * 6 + c, c] = 1.0
    for j2 in range(5):
        for c2 in range(16):
            EP[j2 * 16 + c2, 8 + c2] = 1.0
    for k in range(16):
        EP[k, 24 + k] = 1.0
    return G1, G2, S1, EP


_G1, _G2, _S1, _EP = _patterns()


def _prep_body(ww_ref, bb_ref, g1_ref, g2_ref, s1hot_ref, ep_ref,
               wp_ref, bp_ref):
    f32 = jnp.float32
    bf16 = jnp.bfloat16
    wp_ref[...] = jnp.zeros(wp_ref.shape, bf16)
    bp_ref[...] = jnp.zeros(bp_ref.shape, f32)

    sc1 = bb_ref[0:6, 1:2] * jax.lax.rsqrt(bb_ref[0:6, 4:5] + _EPS)
    b1e = bb_ref[0:6, 0:1] * sc1 + bb_ref[0:6, 2:3] - bb_ref[0:6, 3:4] * sc1
    w1e = (ww_ref[_W_C1:_W_C1 + 6, 0:25] * sc1).astype(bf16)
    sc2 = bb_ref[0:16, 6:7] * jax.lax.rsqrt(bb_ref[0:16, 9:10] + _EPS)
    b2e = bb_ref[0:16, 5:6] * sc2 + bb_ref[0:16, 7:8] - bb_ref[0:16, 8:9] * sc2
    w2e = (ww_ref[_W_C2:_W_C2 + 16, 0:150] * sc2).astype(bf16)

    for d in range(4):
        s = jax.lax.dot(w1e, g1_ref[:, 192 * d:192 * (d + 1)],
                        preferred_element_type=f32).astype(bf16)
        for j in range(14):
            r0 = _R_T1 + d * 96 + j * 6
            wp_ref[r0:r0 + 6, 2 * j:2 * j + 192] = s

    for d in range(4):
        s = jax.lax.dot(w2e, g2_ref[0:150, 576 * d:576 * (d + 1)],
                        preferred_element_type=f32).astype(bf16)
        for j2 in range(5):
            r0 = _R_U2 + d * 80 + j2 * 16
            wp_ref[r0:r0 + 16, 12 * j2:12 * j2 + 576] = s

    fw1 = jax.lax.dot(ww_ref[_W_FC1:_W_FC1 + 120, 0:400].astype(bf16),
                      s1hot_ref[...], preferred_element_type=f32)
    wp_ref[_R_FW1:_R_FW1 + 120, 0:400] = fw1.astype(bf16)
    wp_ref[_R_FW2:_R_FW2 + 84, 0:120] = (
        ww_ref[_W_FC2:_W_FC2 + 84, 0:120].astype(bf16))
    wp_ref[_R_FW3:_R_FW3 + 10, 0:84] = (
        ww_ref[_W_FC3:_W_FC3 + 10, 0:84].astype(bf16))
    wp_ref[_R_FB3:_R_FB3 + 1, 0:16] = jax.lax.dot_general(
        bb_ref[0:10, 10:11], ep_ref[0:10, 24:40],
        dimension_numbers=(((0,), (0,)), ((), ())),
        preferred_element_type=f32).astype(bf16)

    bp_ref[_R_S1:_R_S1 + 96, :] = jax.lax.dot(
        ep_ref[:, 0:6], b1e, preferred_element_type=f32)
    bp_ref[_R_S2:_R_S2 + 80, :] = jax.lax.dot(
        ep_ref[0:80, 8:24], b2e, preferred_element_type=f32)
    bp_ref[_R_FB1:_R_FB1 + 120, :] = bb_ref[0:120, 11:12]
    bp_ref[_R_FB2:_R_FB2 + 84, :] = bb_ref[0:84, 12:13]


def _lenet_body(x_ref, wp_ref, bp_ref, out_ref, p1_ref, a_ref):
    t1 = wp_ref[_R_T1:_R_T1 + 384, 0:192]
    u2 = wp_ref[_R_U2:_R_U2 + 320, 0:576]
    s1 = bp_ref[_R_S1:_R_S1 + 96, :]
    s2 = bp_ref[_R_S2:_R_S2 + 80, :]

    for hh in range(14):
        xr = x_ref[64 * hh:64 * hh + 192, :]
        c = jax.lax.dot(t1, xr, preferred_element_type=jnp.float32)
        m = jnp.maximum(jnp.maximum(c[0:96], c[96:192]),
                        jnp.maximum(c[192:288], c[288:384]))
        p1_ref[96 * hh:96 * hh + 96, :] = (
            jnp.tanh(m + s1).astype(jnp.bfloat16))

    for ii in range(5):
        r = p1_ref[192 * ii:192 * ii + 576, :]
        c = jax.lax.dot(u2, r, preferred_element_type=jnp.float32)
        m = jnp.maximum(jnp.maximum(c[0:80], c[80:160]),
                        jnp.maximum(c[160:240], c[240:320]))
        a_ref[80 * ii:80 * ii + 80, :] = (
            jnp.tanh(m + s2).astype(jnp.bfloat16))

    h = jnp.tanh(jax.lax.dot(wp_ref[_R_FW1:_R_FW1 + 128, 0:400], a_ref[...],
                             preferred_element_type=jnp.float32)
                 + bp_ref[_R_FB1:_R_FB1 + 128, :]).astype(jnp.bfloat16)
    h = jnp.tanh(jax.lax.dot(wp_ref[_R_FW2:_R_FW2 + 128, 0:128], h,
                             preferred_element_type=jnp.float32)
                 + bp_ref[_R_FB2:_R_FB2 + 128, :]).astype(jnp.bfloat16)
    out_ref[...] = (jax.lax.dot_general(
        h, wp_ref[_R_FW3:_R_FW3 + 16, 0:128],
        dimension_numbers=(((0,), (1,)), ((), ())),
        preferred_element_type=jnp.float32)
        + wp_ref[_R_FB3:_R_FB3 + 1, 0:16].astype(jnp.float32))


def kernel(conv1_w, conv1_b, conv2_w, conv2_b,
           bn1_gamma, bn1_beta, bn1_mean, bn1_var,
           bn2_gamma, bn2_beta, bn2_mean, bn2_var,
           fc1_w, fc1_b, fc2_w, fc2_b, fc3_w, fc3_b, img):
    bf16 = jnp.bfloat16
    f32 = jnp.float32

    ww = jnp.concatenate([
        fc1_w,
        jnp.pad(fc2_w, ((0, 0), (0, 280))),
        jnp.pad(fc3_w, ((0, 0), (0, 316))),
        jnp.pad(conv2_w.reshape(16, 150), ((0, 0), (0, 250))),
        jnp.pad(conv1_w.reshape(6, 25), ((0, 0), (0, 375))),
        jnp.zeros((4, 400), f32),
    ], axis=0)
    cols = [conv1_b, bn1_gamma, bn1_beta, bn1_mean, bn1_var,
            conv2_b, bn2_gamma, bn2_beta, bn2_mean, bn2_var,
            fc3_b, fc1_b, fc2_b]
    bb = jnp.stack([jnp.pad(c, (0, 128 - c.shape[0])) for c in cols]
                   + [jnp.zeros(128, f32)] * 3, axis=1)

    full = lambda a: pl.BlockSpec(a.shape, lambda: (0,) * a.ndim)
    prep_in = (ww, bb, jnp.asarray(_G1, bf16), jnp.asarray(_G2, bf16),
               jnp.asarray(_S1, bf16), jnp.asarray(_EP, f32))
    wp, bp = pl.pallas_call(
        _prep_body,
        out_shape=(jax.ShapeDtypeStruct((_WROWS, _WCOLS), bf16),
                   jax.ShapeDtypeStruct((_BROWS, 1), f32)),
        in_specs=[full(a) for a in prep_in],
        out_specs=(pl.BlockSpec((_WROWS, _WCOLS), lambda: (0, 0)),
                   pl.BlockSpec((_BROWS, 1), lambda: (0, 0))),
    )(*prep_in)

    b = img.shape[0]
    b_pad = ((b + _BT - 1) // _BT) * _BT
    x = img.reshape(b, 32 * 32).astype(bf16)
    if b_pad != b:
        x = jnp.pad(x, ((0, b_pad - b), (0, 0)))
    x_t = x.T

    fullg = lambda shape: pl.BlockSpec(shape, lambda i: (0,) * len(shape))
    out = pl.pallas_call(
        _lenet_body,
        out_shape=jax.ShapeDtypeStruct((b_pad, _NC), f32),
        grid=(b_pad // _BT,),
        in_specs=[
            pl.BlockSpec((1024, _BT), lambda i: (0, i)),
            fullg((_WROWS, _WCOLS)), fullg((_BROWS, 1)),
        ],
        out_specs=pl.BlockSpec((_BT, _NC), lambda i: (i, 0)),
        scratch_shapes=[
            pltpu.VMEM((14 * 96, _BT), jnp.bfloat16),
            pltpu.VMEM((400, _BT), jnp.bfloat16),
        ],
        compiler_params=pltpu.CompilerParams(
            dimension_semantics=("arbitrary",)),
    )(x_t, wp, bp)

    return out[:b, :fc3_b.shape[0]]


# speedup vs baseline: 1.0680x; 1.0680x over previous
import numpy as np

import jax
import jax.numpy as jnp
from jax.experimental import pallas as pl
from jax.experimental.pallas import tpu as pltpu

_EPS = 1e-5
_BT = 1024
_NP = 128
_NC = 16

_R_T1, _R_U2, _R_FW1, _R_FW2, _R_FW3, _R_FB3 = 0, 384, 704, 832, 960, 976
_WROWS, _WCOLS = 984, 640
_R_S1, _R_S2, _R_FB1, _R_FB2 = 0, 96, 176, 304
_BROWS = 432
_W_FC1, _W_FC2, _W_FC3, _W_C2, _W_C1 = 0, 120, 204, 214, 230


def _patterns():
    G1 = np.zeros((25, 4 * 192), np.float32)
    G2 = np.zeros((152, 4 * 504), np.float32)
    for kh in range(5):
        for kw in range(5):
            for dr in range(2):
                for dc in range(2):
                    d = dr * 2 + dc
                    G1[kh * 5 + kw, d * 192 + (dr + kh) * 32 + dc + kw] = 1.0
                    for ci in range(6):
                        G2[ci * 25 + kh * 5 + kw,
                           d * 504 + (dr + kh) * 84 + (dc + kw) * 6 + ci] = 1.0

    S1 = np.zeros((400, 400), np.float32)
    for ii in range(5):
        for j2 in range(5):
            for c2 in range(16):
                S1[c2 * 25 + ii * 5 + j2, ii * 80 + j2 * 16 + c2] = 1.0

    EP = np.zeros((96, 40), np.float32)
    for j in range(14):
        for c in range(6):
            EP[j * 6 + c, c] = 1.0
    for j2 in range(5):
        for c2 in range(16):
            EP[j2 * 16 + c2, 8 + c2] = 1.0
    for k in range(16):
        EP[k, 24 + k] = 1.0
    return G1, G2, S1, EP


_G1, _G2, _S1, _EP = _patterns()


def _prep_body(ww_ref, bb_ref, g1_ref, g2_ref, s1hot_ref, ep_ref,
               wp_ref, bp_ref):
    f32 = jnp.float32
    bf16 = jnp.bfloat16
    wp_ref[...] = jnp.zeros(wp_ref.shape, bf16)
    bp_ref[...] = jnp.zeros(bp_ref.shape, f32)

    sc1 = bb_ref[0:6, 1:2] * jax.lax.rsqrt(bb_ref[0:6, 4:5] + _EPS)
    b1e = bb_ref[0:6, 0:1] * sc1 + bb_ref[0:6, 2:3] - bb_ref[0:6, 3:4] * sc1
    w1e = (ww_ref[_W_C1:_W_C1 + 6, 0:25] * sc1).astype(bf16)
    sc2 = bb_ref[0:16, 6:7] * jax.lax.rsqrt(bb_ref[0:16, 9:10] + _EPS)
    b2e = bb_ref[0:16, 5:6] * sc2 + bb_ref[0:16, 7:8] - bb_ref[0:16, 8:9] * sc2
    w2e = (ww_ref[_W_C2:_W_C2 + 16, 0:150] * sc2).astype(bf16)

    for d in range(4):
        s = jax.lax.dot(w1e, g1_ref[:, 192 * d:192 * (d + 1)],
                        preferred_element_type=f32).astype(bf16)
        for j in range(14):
            r0 = _R_T1 + d * 96 + j * 6
            wp_ref[r0:r0 + 6, 2 * j:2 * j + 192] = s

    for d in range(4):
        s = jax.lax.dot(w2e, g2_ref[0:150, 504 * d:504 * (d + 1)],
                        preferred_element_type=f32).astype(bf16)
        for j2 in range(5):
            r0 = _R_U2 + d * 80 + j2 * 16
            wp_ref[r0:r0 + 16, 12 * j2:12 * j2 + 504] = s

    fw1 = jax.lax.dot(ww_ref[_W_FC1:_W_FC1 + 120, 0:400].astype(bf16),
                      s1hot_ref[...], preferred_element_type=f32)
    wp_ref[_R_FW1:_R_FW1 + 120, 0:400] = fw1.astype(bf16)
    wp_ref[_R_FW2:_R_FW2 + 84, 0:120] = (
        ww_ref[_W_FC2:_W_FC2 + 84, 0:120].astype(bf16))
    wp_ref[_R_FW3:_R_FW3 + 10, 0:84] = (
        ww_ref[_W_FC3:_W_FC3 + 10, 0:84].astype(bf16))
    wp_ref[_R_FB3:_R_FB3 + 1, 0:16] = jax.lax.dot_general(
        bb_ref[0:10, 10:11], ep_ref[0:10, 24:40],
        dimension_numbers=(((0,), (0,)), ((), ())),
        preferred_element_type=f32).astype(bf16)

    bp_ref[_R_S1:_R_S1 + 96, :] = jax.lax.dot(
        ep_ref[:, 0:6], b1e, preferred_element_type=f32)
    bp_ref[_R_S2:_R_S2 + 80, :] = jax.lax.dot(
        ep_ref[0:80, 8:24], b2e, preferred_element_type=f32)
    bp_ref[_R_FB1:_R_FB1 + 120, :] = bb_ref[0:120, 11:12]
    bp_ref[_R_FB2:_R_FB2 + 84, :] = bb_ref[0:84, 12:13]


def _lenet_body(x_ref, wp_ref, bp_ref, out_ref, p1_ref, a_ref):
    t1 = wp_ref[_R_T1:_R_T1 + 384, 0:192]
    u2 = wp_ref[_R_U2:_R_U2 + 320, 0:504]
    s1 = bp_ref[_R_S1:_R_S1 + 96, :]
    s2 = bp_ref[_R_S2:_R_S2 + 80, :]

    for hh in range(14):
        xr = x_ref[64 * hh:64 * hh + 192, :]
        c = jax.lax.dot(t1, xr, preferred_element_type=jnp.float32)
        m = jnp.maximum(jnp.maximum(c[0:96], c[96:192]),
                        jnp.maximum(c[192:288], c[288:384]))
        p1_ref[84 * hh:84 * hh + 84, :] = (
            jnp.tanh(m[0:84] + s1[0:84]).astype(jnp.bfloat16))

    for ii in range(5):
        r = p1_ref[168 * ii:168 * ii + 504, :]
        c = jax.lax.dot(u2, r, preferred_element_type=jnp.float32)
        m = jnp.maximum(jnp.maximum(c[0:80], c[80:160]),
                        jnp.maximum(c[160:240], c[240:320]))
        a_ref[80 * ii:80 * ii + 80, :] = (
            jnp.tanh(m + s2).astype(jnp.bfloat16))

    h = jnp.tanh(jax.lax.dot(wp_ref[_R_FW1:_R_FW1 + 128, 0:400], a_ref[...],
                             preferred_element_type=jnp.float32)
                 + bp_ref[_R_FB1:_R_FB1 + 128, :]).astype(jnp.bfloat16)
    h = jnp.tanh(jax.lax.dot(wp_ref[_R_FW2:_R_FW2 + 128, 0:128], h,
                             preferred_element_type=jnp.float32)
                 + bp_ref[_R_FB2:_R_FB2 + 128, :]).astype(jnp.bfloat16)
    out_ref[...] = (jax.lax.dot_general(
        h, wp_ref[_R_FW3:_R_FW3 + 16, 0:128],
        dimension_numbers=(((0,), (1,)), ((), ())),
        preferred_element_type=jnp.float32)
        + wp_ref[_R_FB3:_R_FB3 + 1, 0:16].astype(jnp.float32))


def kernel(conv1_w, conv1_b, conv2_w, conv2_b,
           bn1_gamma, bn1_beta, bn1_mean, bn1_var,
           bn2_gamma, bn2_beta, bn2_mean, bn2_var,
           fc1_w, fc1_b, fc2_w, fc2_b, fc3_w, fc3_b, img):
    bf16 = jnp.bfloat16
    f32 = jnp.float32

    ww = jnp.concatenate([
        fc1_w,
        jnp.pad(fc2_w, ((0, 0), (0, 280))),
        jnp.pad(fc3_w, ((0, 0), (0, 316))),
        jnp.pad(conv2_w.reshape(16, 150), ((0, 0), (0, 250))),
        jnp.pad(conv1_w.reshape(6, 25), ((0, 0), (0, 375))),
        jnp.zeros((4, 400), f32),
    ], axis=0)
    cols = [conv1_b, bn1_gamma, bn1_beta, bn1_mean, bn1_var,
            conv2_b, bn2_gamma, bn2_beta, bn2_mean, bn2_var,
            fc3_b, fc1_b, fc2_b]
    bb = jnp.stack([jnp.pad(c, (0, 128 - c.shape[0])) for c in cols]
                   + [jnp.zeros(128, f32)] * 3, axis=1)

    full = lambda a: pl.BlockSpec(a.shape, lambda: (0,) * a.ndim)
    prep_in = (ww, bb, jnp.asarray(_G1, bf16), jnp.asarray(_G2, bf16),
               jnp.asarray(_S1, bf16), jnp.asarray(_EP, f32))
    wp, bp = pl.pallas_call(
        _prep_body,
        out_shape=(jax.ShapeDtypeStruct((_WROWS, _WCOLS), bf16),
                   jax.ShapeDtypeStruct((_BROWS, 1), f32)),
        in_specs=[full(a) for a in prep_in],
        out_specs=(pl.BlockSpec((_WROWS, _WCOLS), lambda: (0, 0)),
                   pl.BlockSpec((_BROWS, 1), lambda: (0, 0))),
    )(*prep_in)

    b = img.shape[0]
    b_pad = ((b + _BT - 1) // _BT) * _BT
    x = img.reshape(b, 32 * 32).astype(bf16)
    if b_pad != b:
        x = jnp.pad(x, ((0, b_pad - b), (0, 0)))
    x_t = x.T

    fullg = lambda shape: pl.BlockSpec(shape, lambda i: (0,) * len(shape))
    out = pl.pallas_call(
        _lenet_body,
        out_shape=jax.ShapeDtypeStruct((b_pad, _NC), f32),
        grid=(b_pad // _BT,),
        in_specs=[
            pl.BlockSpec((1024, _BT), lambda i: (0, i)),
            fullg((_WROWS, _WCOLS)), fullg((_BROWS, 1)),
        ],
        out_specs=pl.BlockSpec((_BT, _NC), lambda i: (i, 0)),
        scratch_shapes=[
            pltpu.VMEM((14 * 84, _BT), jnp.bfloat16),
            pltpu.VMEM((400, _BT), jnp.bfloat16),
        ],
        compiler_params=pltpu.CompilerParams(
            dimension_semantics=("arbitrary",)),
    )(x_t, wp, bp)

    return out[:b, :fc3_b.shape[0]]


# speedup vs baseline: 1.1038x; 1.0335x over previous
import numpy as np

import jax
import jax.numpy as jnp
from jax.experimental import pallas as pl
from jax.experimental.pallas import tpu as pltpu

_EPS = 1e-5
_BT = 1024
_NP = 128
_NC = 16

_R_T1, _R_U2, _R_FW1, _R_FW2, _R_FW3, _R_FB3 = 0, 384, 704, 832, 960, 976
_WROWS, _WCOLS = 984, 640
_R_S1, _R_S2, _R_FB1, _R_FB2 = 0, 96, 176, 304
_BROWS = 432
_W_FC1, _W_FC2, _W_FC3, _W_C2, _W_C1 = 0, 120, 204, 214, 230


def _patterns():
    G1 = np.zeros((25, 4 * 192), np.float32)
    G2 = np.zeros((152, 4 * 504), np.float32)
    for kh in range(5):
        for kw in range(5):
            for dr in range(2):
                for dc in range(2):
                    d = dr * 2 + dc
                    G1[kh * 5 + kw, d * 192 + (dr + kh) * 32 + dc + kw] = 1.0
                    for ci in range(6):
                        G2[ci * 25 + kh * 5 + kw,
                           d * 504 + (dr + kh) * 84 + (dc + kw) * 6 + ci] = 1.0

    S1 = np.zeros((400, 400), np.float32)
    for ii in range(5):
        for j2 in range(5):
            for c2 in range(16):
                S1[c2 * 25 + ii * 5 + j2, ii * 80 + j2 * 16 + c2] = 1.0

    EP = np.zeros((96, 40), np.float32)
    for j in range(14):
        for c in range(6):
            EP[j * 6 + c, c] = 1.0
    for j2 in range(5):
        for c2 in range(16):
            EP[j2 * 16 + c2, 8 + c2] = 1.0
    for k in range(16):
        EP[k, 24 + k] = 1.0
    return G1, G2, S1, EP


_G1, _G2, _S1, _EP = _patterns()


def _prep(ww_ref, bb_ref, g1_ref, g2_ref, s1hot_ref, ep_ref,
          wp_ref, bp_ref):
    f32 = jnp.float32
    bf16 = jnp.bfloat16
    wp_ref[...] = jnp.zeros(wp_ref.shape, bf16)
    bp_ref[...] = jnp.zeros(bp_ref.shape, f32)

    sc1 = bb_ref[0:6, 1:2] * jax.lax.rsqrt(bb_ref[0:6, 4:5] + _EPS)
    b1e = bb_ref[0:6, 0:1] * sc1 + bb_ref[0:6, 2:3] - bb_ref[0:6, 3:4] * sc1
    w1e = (ww_ref[_W_C1:_W_C1 + 6, 0:25] * sc1).astype(bf16)
    sc2 = bb_ref[0:16, 6:7] * jax.lax.rsqrt(bb_ref[0:16, 9:10] + _EPS)
    b2e = bb_ref[0:16, 5:6] * sc2 + bb_ref[0:16, 7:8] - bb_ref[0:16, 8:9] * sc2
    w2e = (ww_ref[_W_C2:_W_C2 + 16, 0:150] * sc2).astype(bf16)

    for d in range(4):
        s = jax.lax.dot(w1e, g1_ref[:, 192 * d:192 * (d + 1)],
                        preferred_element_type=f32).astype(bf16)
        for j in range(14):
            r0 = _R_T1 + d * 96 + j * 6
            wp_ref[r0:r0 + 6, 2 * j:2 * j + 192] = s

    for d in range(4):
        s = jax.lax.dot(w2e, g2_ref[0:150, 504 * d:504 * (d + 1)],
                        preferred_element_type=f32).astype(bf16)
        for j2 in range(5):
            r0 = _R_U2 + d * 80 + j2 * 16
            wp_ref[r0:r0 + 16, 12 * j2:12 * j2 + 504] = s

    fw1 = jax.lax.dot(ww_ref[_W_FC1:_W_FC1 + 120, 0:400].astype(bf16),
                      s1hot_ref[...], preferred_element_type=f32)
    wp_ref[_R_FW1:_R_FW1 + 120, 0:400] = fw1.astype(bf16)
    wp_ref[_R_FW2:_R_FW2 + 84, 0:120] = (
        ww_ref[_W_FC2:_W_FC2 + 84, 0:120].astype(bf16))
    wp_ref[_R_FW3:_R_FW3 + 10, 0:84] = (
        ww_ref[_W_FC3:_W_FC3 + 10, 0:84].astype(bf16))
    wp_ref[_R_FB3:_R_FB3 + 1, 0:16] = jax.lax.dot_general(
        bb_ref[0:10, 10:11], ep_ref[0:10, 24:40],
        dimension_numbers=(((0,), (0,)), ((), ())),
        preferred_element_type=f32).astype(bf16)

    bp_ref[_R_S1:_R_S1 + 96, :] = jax.lax.dot(
        ep_ref[:, 0:6], b1e, preferred_element_type=f32)
    bp_ref[_R_S2:_R_S2 + 80, :] = jax.lax.dot(
        ep_ref[0:80, 8:24], b2e, preferred_element_type=f32)
    bp_ref[_R_FB1:_R_FB1 + 120, :] = bb_ref[0:120, 11:12]
    bp_ref[_R_FB2:_R_FB2 + 84, :] = bb_ref[0:84, 12:13]


def _lenet_body(x_ref, ww_ref, bb_ref, g1_ref, g2_ref, s1hot_ref, ep_ref,
                out_ref, wp_ref, bp_ref, p1_ref, a_ref):
    @pl.when(pl.program_id(0) == 0)
    def _run_prep():
        _prep(ww_ref, bb_ref, g1_ref, g2_ref, s1hot_ref, ep_ref,
              wp_ref, bp_ref)

    t1 = wp_ref[_R_T1:_R_T1 + 384, 0:192]
    u2 = wp_ref[_R_U2:_R_U2 + 320, 0:504]
    s1 = bp_ref[_R_S1:_R_S1 + 96, :]
    s2 = bp_ref[_R_S2:_R_S2 + 80, :]

    for hh in range(14):
        xr = x_ref[64 * hh:64 * hh + 192, :]
        c = jax.lax.dot(t1, xr, preferred_element_type=jnp.float32)
        m = jnp.maximum(jnp.maximum(c[0:96], c[96:192]),
                        jnp.maximum(c[192:288], c[288:384]))
        p1_ref[84 * hh:84 * hh + 84, :] = (
            jnp.tanh(m[0:84] + s1[0:84]).astype(jnp.bfloat16))

    for ii in range(5):
        r = p1_ref[168 * ii:168 * ii + 504, :]
        c = jax.lax.dot(u2, r, preferred_element_type=jnp.float32)
        m = jnp.maximum(jnp.maximum(c[0:80], c[80:160]),
                        jnp.maximum(c[160:240], c[240:320]))
        a_ref[80 * ii:80 * ii + 80, :] = (
            jnp.tanh(m + s2).astype(jnp.bfloat16))

    h = jnp.tanh(jax.lax.dot(wp_ref[_R_FW1:_R_FW1 + 128, 0:400], a_ref[...],
                             preferred_element_type=jnp.float32)
                 + bp_ref[_R_FB1:_R_FB1 + 128, :]).astype(jnp.bfloat16)
    h = jnp.tanh(jax.lax.dot(wp_ref[_R_FW2:_R_FW2 + 128, 0:128], h,
                             preferred_element_type=jnp.float32)
                 + bp_ref[_R_FB2:_R_FB2 + 128, :]).astype(jnp.bfloat16)
    out_ref[...] = (jax.lax.dot_general(
        h, wp_ref[_R_FW3:_R_FW3 + 16, 0:128],
        dimension_numbers=(((0,), (1,)), ((), ())),
        preferred_element_type=jnp.float32)
        + wp_ref[_R_FB3:_R_FB3 + 1, 0:16].astype(jnp.float32))


def kernel(conv1_w, conv1_b, conv2_w, conv2_b,
           bn1_gamma, bn1_beta, bn1_mean, bn1_var,
           bn2_gamma, bn2_beta, bn2_mean, bn2_var,
           fc1_w, fc1_b, fc2_w, fc2_b, fc3_w, fc3_b, img):
    bf16 = jnp.bfloat16
    f32 = jnp.float32

    ww = jnp.concatenate([
        fc1_w,
        jnp.pad(fc2_w, ((0, 0), (0, 280))),
        jnp.pad(fc3_w, ((0, 0), (0, 316))),
        jnp.pad(conv2_w.reshape(16, 150), ((0, 0), (0, 250))),
        jnp.pad(conv1_w.reshape(6, 25), ((0, 0), (0, 375))),
        jnp.zeros((4, 400), f32),
    ], axis=0)
    cols = [conv1_b, bn1_gamma, bn1_beta, bn1_mean, bn1_var,
            conv2_b, bn2_gamma, bn2_beta, bn2_mean, bn2_var,
            fc3_b, fc1_b, fc2_b]
    bb = jnp.stack([jnp.pad(c, (0, 128 - c.shape[0])) for c in cols]
                   + [jnp.zeros(128, f32)] * 3, axis=1)

    b = img.shape[0]
    b_pad = ((b + _BT - 1) // _BT) * _BT
    x = img.reshape(b, 32 * 32).astype(bf16)
    if b_pad != b:
        x = jnp.pad(x, ((0, b_pad - b), (0, 0)))
    x_t = x.T

    consts = (ww, bb, jnp.asarray(_G1, bf16), jnp.asarray(_G2, bf16),
              jnp.asarray(_S1, bf16), jnp.asarray(_EP, f32))
    fullg = lambda a: pl.BlockSpec(a.shape, lambda i: (0,) * a.ndim)
    out = pl.pallas_call(
        _lenet_body,
        out_shape=jax.ShapeDtypeStruct((b_pad, _NC), f32),
        grid=(b_pad // _BT,),
        in_specs=[pl.BlockSpec((1024, _BT), lambda i: (0, i))]
                 + [fullg(a) for a in consts],
        out_specs=pl.BlockSpec((_BT, _NC), lambda i: (i, 0)),
        scratch_shapes=[
            pltpu.VMEM((_WROWS, _WCOLS), jnp.bfloat16),
            pltpu.VMEM((_BROWS, 1), jnp.float32),
            pltpu.VMEM((14 * 84, _BT), jnp.bfloat16),
            pltpu.VMEM((400, _BT), jnp.bfloat16),
        ],
        compiler_params=pltpu.CompilerParams(
            dimension_semantics=("arbitrary",)),
    )(x_t, *consts)

    return out[:b, :fc3_b.shape[0]]
